# Initial kernel scaffold; baseline (speedup 1.0000x reference)
import jax
import jax.numpy as jnp
from jax.experimental import pallas as pl


def kernel(nodes, times, mem_data, mem_time, mailbox_mail, mailbox_time, nfeat, efeat, W_ih, W_hh, b_ih, b_hh, te_w, te_b, tl_w, tl_b, ln_g, ln_b, ep_src_w, ep_src_b, ep_dst_w, ep_dst_b, ep_out_w, ep_out_b):
    raise NotImplementedError("write your pallas kernel here")


# speedup vs baseline: 1.3883x; 1.3883x over previous
import functools

import jax
import jax.numpy as jnp
from jax import lax
from jax.experimental import pallas as pl
from jax.experimental.pallas import tpu as pltpu
from jax.experimental.pallas import tpu_sc as plsc

M = 100000
S = 16384
N = 3 * S
L2 = 2 * S
DE = 128
DEDGE = 16
DT = 16
DM = DE + DEDGE

NW = 32
GPW = N // NW
GCH = GPW // 128

_VMESH = plsc.VectorSubcoreMesh(core_axis_name="c", subcore_axis_name="s")


def _f32(*shape):
    return jax.ShapeDtypeStruct(shape, jnp.float32)


def _i32(*shape):
    return jax.ShapeDtypeStruct(shape, jnp.int32)


def _pack_times(mem_time, mailbox_time):
    def body(a_ref, b_ref, o_ref):
        a = a_ref[...][:, None]
        b = b_ref[...][:, None]
        z = jnp.zeros((a.shape[0], 14), jnp.float32)
        o_ref[...] = jnp.concatenate([a, b, z], axis=1)

    blk = 4096
    grid = (M + blk - 1) // blk
    return pl.pallas_call(
        body,
        grid=(grid,),
        in_specs=[
            pl.BlockSpec((blk,), lambda i: (i,)),
            pl.BlockSpec((blk,), lambda i: (i,)),
        ],
        out_specs=pl.BlockSpec((blk, 16), lambda i: (i, 0)),
        out_shape=_f32(M, 16),
    )(mem_time, mailbox_time)


def _sc_gather(nodes2d, P, mem_data, mailbox_mail, nfeat):
    @functools.partial(
        pl.kernel,
        mesh=_VMESH,
        compiler_params=pltpu.CompilerParams(use_tc_tiling_on_sc=False),
        out_type=[_f32(N, 16), _f32(N, DE), _f32(N, DM), _f32(N, DE)],
        scratch_types=[
            pltpu.VMEM((GCH, 128), jnp.int32),
            pltpu.VMEM((128, 16), jnp.float32),
            pltpu.VMEM((128, DE), jnp.float32),
            pltpu.VMEM((128, DM), jnp.float32),
            pltpu.VMEM((128, DE), jnp.float32),
            pltpu.SemaphoreType.DMA,
        ],
    )
    def k(nodes_hbm, p_hbm, mem_hbm, mail_hbm, nf_hbm,
          gp_hbm, gmem_hbm, gmail_hbm, gnf_hbm,
          idx_v, bufp, bufmem, bufmail, bufnf, sem):
        wid = lax.axis_index("s") * 2 + lax.axis_index("c")
        pltpu.sync_copy(nodes_hbm.at[wid], idx_v)

        @pl.loop(0, GCH)
        def _(c):
            row0 = wid * GPW + c * 128
            h1 = pltpu.async_copy(p_hbm.at[idx_v.at[c]], bufp, sem)
            h2 = pltpu.async_copy(mem_hbm.at[idx_v.at[c]], bufmem, sem)
            h3 = pltpu.async_copy(mail_hbm.at[idx_v.at[c]], bufmail, sem)
            h4 = pltpu.async_copy(nf_hbm.at[idx_v.at[c]], bufnf, sem)
            h1.wait(); h2.wait(); h3.wait(); h4.wait()
            pltpu.sync_copy(bufp, gp_hbm.at[pl.ds(row0, 128)])
            pltpu.sync_copy(bufmem, gmem_hbm.at[pl.ds(row0, 128)])
            pltpu.sync_copy(bufmail, gmail_hbm.at[pl.ds(row0, 128)])
            pltpu.sync_copy(bufnf, gnf_hbm.at[pl.ds(row0, 128)])

    return k(nodes2d, P, mem_data, mailbox_mail, nfeat)


def _tc_dense(gp, gmail, gmem, gnf, times,
              W_ih, W_hh, b_ih, b_hh, te_w, te_b, tl_wT, tl_b, ln_g, ln_b):
    B = 512

    def body(gp_ref, gmail_ref, gmem_ref, gnf_ref, t_ref,
             wih_ref, whh_ref, bih_ref, bhh_ref, tew_ref, teb_ref,
             tlw_ref, tlb_ref, lng_ref, lnb_ref, emb_ref, proj_ref):
        gp_b = gp_ref[...]
        mem_ts = gp_b[:, 0]
        mail_ts = gp_b[:, 1]
        dt = mail_ts - mem_ts
        tf = jnp.cos(dt[:, None] * tew_ref[...] + teb_ref[...])
        inp = jnp.concatenate([gmail_ref[...], tf], axis=1)
        cdims = (((1,), (1,)), ((), ()))
        pre = lax.dot_general(inp, wih_ref[...], cdims,
                              preferred_element_type=jnp.float32)
        pre = pre + lax.dot_general(gmem_ref[...], whh_ref[...], cdims,
                                    preferred_element_type=jnp.float32)
        pre = pre + bih_ref[...] + bhh_ref[...]
        emb = jnp.tanh(pre) + gnf_ref[...]
        mu = jnp.mean(emb, axis=1, keepdims=True)
        var = jnp.mean((emb - mu) ** 2, axis=1, keepdims=True)
        emb = (emb - mu) / jnp.sqrt(var + 1e-5) * lng_ref[...] + lnb_ref[...]
        emb_ref[...] = emb
        t = t_ref[...]
        td = (t - mail_ts) / (t + 1.0)
        proj_ref[...] = emb * (1.0 + td[:, None] * tlw_ref[...] + tlb_ref[...])

    full = lambda r, c: pl.BlockSpec((r, c), lambda i: (0, 0))
    return pl.pallas_call(
        body,
        grid=(N // B,),
        in_specs=[
            pl.BlockSpec((B, 16), lambda i: (i, 0)),
            pl.BlockSpec((B, DM), lambda i: (i, 0)),
            pl.BlockSpec((B, DE), lambda i: (i, 0)),
            pl.BlockSpec((B, DE), lambda i: (i, 0)),
            pl.BlockSpec((B,), lambda i: (i,)),
            full(DE, DM + DT), full(DE, DE), full(1, DE), full(1, DE),
            full(1, DT), full(1, DT), full(1, DE), full(1, DE),
            full(1, DE), full(1, DE),
        ],
        out_specs=[
            pl.BlockSpec((B, DE), lambda i: (i, 0)),
            pl.BlockSpec((B, DE), lambda i: (i, 0)),
        ],
        out_shape=[_f32(N, DE), _f32(N, DE)],
    )(gp, gmail, gmem, gnf, times,
      W_ih, W_hh, b_ih, b_hh, te_w, te_b, tl_wT, tl_b, ln_g, ln_b)


def _tc_scores(proj, ep_src_w, ep_src_b, ep_dst_w, ep_dst_b,
               ep_out_w, ep_out_b):
    B = 512

    def body(ps_ref, pd_ref, pn_ref, sw_ref, sb_ref, dw_ref, db_ref,
             ow_ref, ob_ref, pos_ref, neg_ref):
        cdims = (((1,), (1,)), ((), ()))
        hs = lax.dot_general(ps_ref[...], sw_ref[...], cdims,
                             preferred_element_type=jnp.float32) + sb_ref[...]
        hd = lax.dot_general(pd_ref[...], dw_ref[...], cdims,
                             preferred_element_type=jnp.float32) + db_ref[...]
        hn = lax.dot_general(pn_ref[...], dw_ref[...], cdims,
                             preferred_element_type=jnp.float32) + db_ref[...]
        ow = ow_ref[...]
        ob = ob_ref[0, 0]
        pos_ref[...] = jnp.sum(jax.nn.relu(hs + hd) * ow, axis=1) + ob
        neg_ref[...] = jnp.sum(jax.nn.relu(hs + hn) * ow, axis=1) + ob

    nb = S // B
    full = lambda r, c: pl.BlockSpec((r, c), lambda i: (0, 0))
    return pl.pallas_call(
        body,
        grid=(nb,),
        in_specs=[
            pl.BlockSpec((B, DE), lambda i: (i, 0)),
            pl.BlockSpec((B, DE), lambda i, _nb=nb: (i + _nb, 0)),
            pl.BlockSpec((B, DE), lambda i, _nb=nb: (i + 2 * _nb, 0)),
            full(DE, DE), full(1, DE), full(DE, DE), full(1, DE),
            full(1, DE), full(1, 1),
        ],
        out_specs=[
            pl.BlockSpec((B,), lambda i: (i,)),
            pl.BlockSpec((B,), lambda i: (i,)),
        ],
        out_shape=[_f32(S), _f32(S)],
    )(proj, proj, proj, ep_src_w, ep_src_b, ep_dst_w, ep_dst_b,
      ep_out_w, ep_out_b)


def kernel(nodes, times, mem_data, mem_time, mailbox_mail, mailbox_time,
           nfeat, efeat, W_ih, W_hh, b_ih, b_hh, te_w, te_b, tl_w, tl_b,
           ln_g, ln_b, ep_src_w, ep_src_b, ep_dst_w, ep_dst_b,
           ep_out_w, ep_out_b):
    P = _pack_times(mem_time, mailbox_time)
    nodes2d = nodes.reshape(NW, GCH, 128)
    gp, gmem, gmail, gnf = _sc_gather(nodes2d, P, mem_data, mailbox_mail,
                                      nfeat)
    embed, proj = _tc_dense(
        gp, gmail, gmem, gnf, times,
        W_ih, W_hh, b_ih.reshape(1, DE), b_hh.reshape(1, DE),
        te_w.reshape(1, DT), te_b.reshape(1, DT),
        tl_w.reshape(1, DE), tl_b.reshape(1, DE),
        ln_g.reshape(1, DE), ln_b.reshape(1, DE))
    pos, neg = _tc_scores(proj, ep_src_w, ep_src_b.reshape(1, DE),
                          ep_dst_w, ep_dst_b.reshape(1, DE),
                          ep_out_w.reshape(1, DE), ep_out_b.reshape(1, 1))

    mail_ts = gp[:, 1]
    upd = nodes[:L2]
    new_mem_data = mem_data.at[upd].set(embed[:L2])
    new_mem_time = mem_time.at[upd].set(mail_ts[:L2])
    src = nodes[:S]
    dst = nodes[S:L2]
    mail_new = jnp.concatenate([new_mem_data[src], efeat], axis=1)
    new_mailbox_mail = mailbox_mail.at[dst].set(mail_new)
    new_mailbox_time = mailbox_time.at[dst].set(times[S:L2])

    return (pos[:, None], neg[:, None], new_mem_data, new_mem_time,
            new_mailbox_mail, new_mailbox_time)


# speedup vs baseline: 1.7728x; 1.2769x over previous
import dataclasses
import functools

import jax
import jax.numpy as jnp
from jax import lax
from jax.experimental import pallas as pl
from jax.experimental.pallas import tpu as pltpu
from jax.experimental.pallas import tpu_sc as plsc

M = 100000
S = 16384
N = 3 * S
L2 = 2 * S
DE = 128
DEDGE = 16
DT = 16
DM = DE + DEDGE

NW = 32
GPW = N // NW
GCH = GPW // 128

_VMESH = plsc.VectorSubcoreMesh(core_axis_name="c", subcore_axis_name="s")

_SC_PARAMS = pltpu.CompilerParams(use_tc_tiling_on_sc=False)
if "needs_layout_passes" in pltpu.CompilerParams.__dataclass_fields__:
    _SC_PARAMS = dataclasses.replace(_SC_PARAMS, needs_layout_passes=False)


def _f32(*shape):
    return jax.ShapeDtypeStruct(shape, jnp.float32)


def _i32(*shape):
    return jax.ShapeDtypeStruct(shape, jnp.int32)


def _pack_times(mem_time, mailbox_time):
    def body(a_ref, b_ref, o_ref):
        a = a_ref[...][:, None]
        b = b_ref[...][:, None]
        z = jnp.zeros((a.shape[0], 14), jnp.float32)
        o_ref[...] = jnp.concatenate([a, b, z], axis=1)

    blk = 4096
    grid = (M + blk - 1) // blk
    return pl.pallas_call(
        body,
        grid=(grid,),
        in_specs=[
            pl.BlockSpec((blk,), lambda i: (i,)),
            pl.BlockSpec((blk,), lambda i: (i,)),
        ],
        out_specs=pl.BlockSpec((blk, 16), lambda i: (i, 0)),
        out_shape=_f32(M, 16),
    )(mem_time, mailbox_time)


def _sc_gather(nodes2d, P, mem_data, mailbox_mail, nfeat):
    @functools.partial(
        pl.kernel,
        mesh=_VMESH,
        compiler_params=_SC_PARAMS,
        out_type=[_f32(N, 16), _f32(N, DE), _f32(N, DM), _f32(N, DE)],
        scratch_types=[
            pltpu.VMEM((GCH, 128), jnp.int32),
            pltpu.VMEM((128, 16), jnp.float32),
            pltpu.VMEM((128, DE), jnp.float32),
            pltpu.VMEM((128, DM), jnp.float32),
            pltpu.VMEM((128, DE), jnp.float32),
            pltpu.SemaphoreType.DMA,
        ],
    )
    def k(nodes_hbm, p_hbm, mem_hbm, mail_hbm, nf_hbm,
          gp_hbm, gmem_hbm, gmail_hbm, gnf_hbm,
          idx_v, bufp, bufmem, bufmail, bufnf, sem):
        wid = lax.axis_index("s") * 2 + lax.axis_index("c")
        pltpu.sync_copy(nodes_hbm.at[wid], idx_v)

        @pl.loop(0, GCH)
        def _(c):
            row0 = wid * GPW + c * 128
            h1 = pltpu.async_copy(p_hbm.at[idx_v.at[c]], bufp, sem)
            h2 = pltpu.async_copy(mem_hbm.at[idx_v.at[c]], bufmem, sem)
            h3 = pltpu.async_copy(mail_hbm.at[idx_v.at[c]], bufmail, sem)
            h4 = pltpu.async_copy(nf_hbm.at[idx_v.at[c]], bufnf, sem)
            h1.wait(); h2.wait(); h3.wait(); h4.wait()
            pltpu.sync_copy(bufp, gp_hbm.at[pl.ds(row0, 128)])
            pltpu.sync_copy(bufmem, gmem_hbm.at[pl.ds(row0, 128)])
            pltpu.sync_copy(bufmail, gmail_hbm.at[pl.ds(row0, 128)])
            pltpu.sync_copy(bufnf, gnf_hbm.at[pl.ds(row0, 128)])

    return k(nodes2d, P, mem_data, mailbox_mail, nfeat)


def _tc_dense(gp, gmail, gmem, gnf, times,
              W_ih, W_hh, b_ih, b_hh, te_w, te_b, tl_wT, tl_b, ln_g, ln_b):
    B = 512

    def body(gp_ref, gmail_ref, gmem_ref, gnf_ref, t_ref,
             wih_ref, whh_ref, bih_ref, bhh_ref, tew_ref, teb_ref,
             tlw_ref, tlb_ref, lng_ref, lnb_ref, emb_ref, proj_ref):
        gp_b = gp_ref[...]
        mem_ts = gp_b[:, 0]
        mail_ts = gp_b[:, 1]
        dt = mail_ts - mem_ts
        tf = jnp.cos(dt[:, None] * tew_ref[...] + teb_ref[...])
        inp = jnp.concatenate([gmail_ref[...], tf], axis=1)
        cdims = (((1,), (1,)), ((), ()))
        pre = lax.dot_general(inp, wih_ref[...], cdims,
                              preferred_element_type=jnp.float32)
        pre = pre + lax.dot_general(gmem_ref[...], whh_ref[...], cdims,
                                    preferred_element_type=jnp.float32)
        pre = pre + bih_ref[...] + bhh_ref[...]
        emb = jnp.tanh(pre) + gnf_ref[...]
        mu = jnp.mean(emb, axis=1, keepdims=True)
        var = jnp.mean((emb - mu) ** 2, axis=1, keepdims=True)
        emb = (emb - mu) / jnp.sqrt(var + 1e-5) * lng_ref[...] + lnb_ref[...]
        emb_ref[...] = emb
        t = t_ref[...]
        td = (t - mail_ts) / (t + 1.0)
        proj_ref[...] = emb * (1.0 + td[:, None] * tlw_ref[...] + tlb_ref[...])

    full = lambda r, c: pl.BlockSpec((r, c), lambda i: (0, 0))
    return pl.pallas_call(
        body,
        grid=(N // B,),
        in_specs=[
            pl.BlockSpec((B, 16), lambda i: (i, 0)),
            pl.BlockSpec((B, DM), lambda i: (i, 0)),
            pl.BlockSpec((B, DE), lambda i: (i, 0)),
            pl.BlockSpec((B, DE), lambda i: (i, 0)),
            pl.BlockSpec((B,), lambda i: (i,)),
            full(DE, DM + DT), full(DE, DE), full(1, DE), full(1, DE),
            full(1, DT), full(1, DT), full(1, DE), full(1, DE),
            full(1, DE), full(1, DE),
        ],
        out_specs=[
            pl.BlockSpec((B, DE), lambda i: (i, 0)),
            pl.BlockSpec((B, DE), lambda i: (i, 0)),
        ],
        out_shape=[_f32(N, DE), _f32(N, DE)],
    )(gp, gmail, gmem, gnf, times,
      W_ih, W_hh, b_ih, b_hh, te_w, te_b, tl_wT, tl_b, ln_g, ln_b)


def _tc_scores(proj, ep_src_w, ep_src_b, ep_dst_w, ep_dst_b,
               ep_out_w, ep_out_b):
    B = 512

    def body(ps_ref, pd_ref, pn_ref, sw_ref, sb_ref, dw_ref, db_ref,
             ow_ref, ob_ref, pos_ref, neg_ref):
        cdims = (((1,), (1,)), ((), ()))
        hs = lax.dot_general(ps_ref[...], sw_ref[...], cdims,
                             preferred_element_type=jnp.float32) + sb_ref[...]
        hd = lax.dot_general(pd_ref[...], dw_ref[...], cdims,
                             preferred_element_type=jnp.float32) + db_ref[...]
        hn = lax.dot_general(pn_ref[...], dw_ref[...], cdims,
                             preferred_element_type=jnp.float32) + db_ref[...]
        ow = ow_ref[...]
        ob = ob_ref[0, 0]
        pos_ref[...] = jnp.sum(jax.nn.relu(hs + hd) * ow, axis=1) + ob
        neg_ref[...] = jnp.sum(jax.nn.relu(hs + hn) * ow, axis=1) + ob

    nb = S // B
    full = lambda r, c: pl.BlockSpec((r, c), lambda i: (0, 0))
    return pl.pallas_call(
        body,
        grid=(nb,),
        in_specs=[
            pl.BlockSpec((B, DE), lambda i: (i, 0)),
            pl.BlockSpec((B, DE), lambda i, _nb=nb: (i + _nb, 0)),
            pl.BlockSpec((B, DE), lambda i, _nb=nb: (i + 2 * _nb, 0)),
            full(DE, DE), full(1, DE), full(DE, DE), full(1, DE),
            full(1, DE), full(1, 1),
        ],
        out_specs=[
            pl.BlockSpec((B,), lambda i: (i,)),
            pl.BlockSpec((B,), lambda i: (i,)),
        ],
        out_shape=[_f32(S), _f32(S)],
    )(proj, proj, proj, ep_src_w, ep_src_b, ep_dst_w, ep_dst_b,
      ep_out_w, ep_out_b)


RANGE = 3200
NCH = RANGE // 128

_IOTA = lambda: lax.broadcasted_iota(jnp.int32, (16,), 0)


def _build_winners(idx_hbm, val_hbm, nbig, base, pos_t, time_t, icbuf, tcbuf):
    @pl.loop(0, nbig)
    def _(bc):
        pltpu.sync_copy(idx_hbm.at[bc], icbuf)
        pltpu.sync_copy(val_hbm.at[bc], tcbuf)

        @pl.loop(0, 64)
        def _(k):
            v = icbuf[pl.ds(k * 16, 16)]
            ts = tcbuf[pl.ds(k * 16, 16)]
            j = bc * 1024 + k * 16 + _IOTA()
            inr = (v >= base) & (v < base + RANGE)
            rel = jnp.clip(v - base, 0, RANGE - 1)
            plsc.store_scatter(pos_t, [rel], j, mask=inr)
            g = plsc.load_gather(pos_t, [rel])

            def fix(bad):
                plsc.store_scatter(pos_t, [rel], j, mask=bad)
                return inr & (j > plsc.load_gather(pos_t, [rel]))

            lax.while_loop(jnp.any, fix, inr & (j > g))
            win = inr & (plsc.load_gather(pos_t, [rel]) == j)
            plsc.store_scatter(time_t, [rel], ts, mask=win)


def _compact(tab, base, rlist, plist, add_base):
    def step(k, cnt):
        tp = tab[pl.ds(k * 16, 16)]
        msk = tp >= 0
        r = base * add_base + k * 16 + _IOTA()
        pos = cnt + plsc.cumsum(msk.astype(jnp.int32)) - 1
        plsc.store_scatter(rlist, [pos // 128, pos % 128], r, mask=msk)
        plsc.store_scatter(plist, [pos // 128, pos % 128], tp, mask=msk)
        return cnt + jnp.sum(msk.astype(jnp.int32))

    cnt = lax.fori_loop(0, RANGE // 16, step, 0)
    pad_end = ((cnt + 127) // 128) * 128
    last = jnp.maximum(cnt - 1, 0)
    li = jnp.full((16,), last // 128, jnp.int32)
    lj = jnp.full((16,), last % 128, jnp.int32)
    rlast = plsc.load_gather(rlist, [li, lj])
    plast = plsc.load_gather(plist, [li, lj])
    for r8 in range(8):
        p = cnt + r8 * 16 + _IOTA()
        pm = p < pad_end
        plsc.store_scatter(rlist, [p // 128, p % 128], rlast, mask=pm)
        plsc.store_scatter(plist, [p // 128, p % 128], plast, mask=pm)
    return pad_end // 128


def _sc_mem_update(upd2d, mts2d, dst2d, tdst2d, embed):
    @functools.partial(
        pl.kernel,
        mesh=_VMESH,
        compiler_params=_SC_PARAMS,
        out_type=[_f32(M, DE), _f32(S, DE), _i32(NW, RANGE),
                  _f32(NW, RANGE), _i32(NW, RANGE), _f32(NW, RANGE)],
        scratch_types=[
            pltpu.VMEM((RANGE,), jnp.int32),
            pltpu.VMEM((RANGE,), jnp.float32),
            pltpu.VMEM((RANGE,), jnp.int32),
            pltpu.VMEM((RANGE,), jnp.float32),
            pltpu.VMEM((1024,), jnp.int32),
            pltpu.VMEM((1024,), jnp.float32),
            pltpu.VMEM((NCH, 128), jnp.int32),
            pltpu.VMEM((NCH, 128), jnp.int32),
            pltpu.VMEM((128, 128), jnp.int32),
            pltpu.VMEM((128, 128), jnp.int32),
            pltpu.VMEM((128, DE), jnp.float32),
            pltpu.SemaphoreType.DMA,
        ],
    )
    def k(upd_hbm, mts_hbm, dst_hbm, tdst_hbm, embed_hbm,
          newmem_hbm, mailnew_hbm, tpos_hbm, ttime_hbm, t2pos_hbm,
          t2time_hbm, tpos_v, ttime_v, t2pos_v, t2time_v, icbuf, tcbuf,
          vlist, jlist, ilist, mjlist, rowbuf, sem):
        wid = lax.axis_index("s") * 2 + lax.axis_index("c")
        base = wid * RANGE

        @pl.loop(0, RANGE // 16)
        def _(k0):
            neg = jnp.full((16,), -1, jnp.int32)
            tpos_v[pl.ds(k0 * 16, 16)] = neg
            t2pos_v[pl.ds(k0 * 16, 16)] = neg

        _build_winners(upd_hbm, mts_hbm, L2 // 1024, base,
                       tpos_v, ttime_v, icbuf, tcbuf)
        _build_winners(dst_hbm, tdst_hbm, S // 1024, base,
                       t2pos_v, t2time_v, icbuf, tcbuf)

        pltpu.sync_copy(tpos_v, tpos_hbm.at[wid])
        pltpu.sync_copy(ttime_v, ttime_hbm.at[wid])
        pltpu.sync_copy(t2pos_v, t2pos_hbm.at[wid])
        pltpu.sync_copy(t2time_v, t2time_hbm.at[wid])


        nchunks = _compact(tpos_v, base, vlist, jlist, 1)

        def scat(cc, _):
            pltpu.async_copy(embed_hbm.at[jlist.at[cc]], rowbuf, sem).wait()
            pltpu.async_copy(rowbuf, newmem_hbm.at[vlist.at[cc]], sem).wait()
            return 0

        lax.fori_loop(0, nchunks, scat, 0)

        def outer(bc, cnt2):
            pltpu.sync_copy(upd_hbm.at[bc], icbuf)

            def inner(k1, cnt):
                v = icbuf[pl.ds(k1 * 16, 16)]
                i = bc * 1024 + k1 * 16 + _IOTA()
                inr = (v >= base) & (v < base + RANGE)
                rel = jnp.clip(v - base, 0, RANGE - 1)
                jj = plsc.load_gather(tpos_v, [rel])
                pos = cnt + plsc.cumsum(inr.astype(jnp.int32)) - 1
                plsc.store_scatter(ilist, [pos // 128, pos % 128], i,
                                   mask=inr)
                plsc.store_scatter(mjlist, [pos // 128, pos % 128], jj,
                                   mask=inr)
                return cnt + jnp.sum(inr.astype(jnp.int32))

            return lax.fori_loop(0, 64, inner, cnt2)

        cnt2 = lax.fori_loop(0, S // 1024, outer, 0)
        pad_end = ((cnt2 + 127) // 128) * 128
        last = jnp.maximum(cnt2 - 1, 0)
        li = jnp.full((16,), last // 128, jnp.int32)
        lj = jnp.full((16,), last % 128, jnp.int32)
        ilast = plsc.load_gather(ilist, [li, lj])
        mjlast = plsc.load_gather(mjlist, [li, lj])
        for r8 in range(8):
            p = cnt2 + r8 * 16 + _IOTA()
            pm = p < pad_end
            plsc.store_scatter(ilist, [p // 128, p % 128], ilast, mask=pm)
            plsc.store_scatter(mjlist, [p // 128, p % 128], mjlast, mask=pm)

        def route(cc, _):
            pltpu.async_copy(embed_hbm.at[mjlist.at[cc]], rowbuf, sem).wait()
            pltpu.async_copy(rowbuf, mailnew_hbm.at[ilist.at[cc]],
                             sem).wait()
            return 0

        lax.fori_loop(0, pad_end // 128, route, 0)

    return k(upd2d, mts2d, dst2d, tdst2d, embed)


def _sc_mail_update(mailnew, efeat, t2pos):
    @functools.partial(
        pl.kernel,
        mesh=_VMESH,
        compiler_params=_SC_PARAMS,
        out_type=[_f32(M, DE), _f32(M, DEDGE)],
        scratch_types=[
            pltpu.VMEM((RANGE,), jnp.int32),
            pltpu.VMEM((NCH, 128), jnp.int32),
            pltpu.VMEM((NCH, 128), jnp.int32),
            pltpu.VMEM((128, DE), jnp.float32),
            pltpu.VMEM((128, DEDGE), jnp.float32),
            pltpu.SemaphoreType.DMA,
        ],
    )
    def k(mailnew_hbm, ef_hbm, t2pos_hbm, out128_hbm, out16_hbm,
          t2loc, dlist, iwlist, bufm, bufe, sem):
        wid = lax.axis_index("s") * 2 + lax.axis_index("c")
        base = wid * RANGE
        pltpu.sync_copy(t2pos_hbm.at[wid], t2loc)
        nchunks = _compact(t2loc, base, dlist, iwlist, 1)

        def scat(cc, _):
            h1 = pltpu.async_copy(mailnew_hbm.at[iwlist.at[cc]], bufm, sem)
            h2 = pltpu.async_copy(ef_hbm.at[iwlist.at[cc]], bufe, sem)
            h1.wait()
            h2.wait()
            h3 = pltpu.async_copy(bufm, out128_hbm.at[dlist.at[cc]], sem)
            h4 = pltpu.async_copy(bufe, out16_hbm.at[dlist.at[cc]], sem)
            h3.wait()
            h4.wait()
            return 0

        lax.fori_loop(0, nchunks, scat, 0)

    return k(mailnew, efeat, t2pos)


def _tc_merge_rows(tab_map, scattered, old, width):
    blk = 4096

    def body(mp_ref, sc_ref, old_ref, o_ref):
        upd = mp_ref[...] >= 0
        o_ref[...] = jnp.where(upd, sc_ref[...], old_ref[...])

    return pl.pallas_call(
        body,
        grid=(pl.cdiv(M, blk),),
        in_specs=[
            pl.BlockSpec((blk, 1), lambda i: (i, 0)),
            pl.BlockSpec((blk, width), lambda i: (i, 0)),
            pl.BlockSpec((blk, width), lambda i: (i, 0)),
        ],
        out_specs=pl.BlockSpec((blk, width), lambda i: (i, 0)),
        out_shape=_f32(M, width),
    )(tab_map, scattered, old)


def _tc_merge_mail(tab_map, sc128, sc16, old):
    blk = 4096

    def body(mp_ref, s1_ref, s2_ref, old_ref, o_ref):
        upd = mp_ref[...] >= 0
        new = jnp.concatenate([s1_ref[...], s2_ref[...]], axis=1)
        o_ref[...] = jnp.where(upd, new, old_ref[...])

    return pl.pallas_call(
        body,
        grid=(pl.cdiv(M, blk),),
        in_specs=[
            pl.BlockSpec((blk, 1), lambda i: (i, 0)),
            pl.BlockSpec((blk, DE), lambda i: (i, 0)),
            pl.BlockSpec((blk, DEDGE), lambda i: (i, 0)),
            pl.BlockSpec((blk, DM), lambda i: (i, 0)),
        ],
        out_specs=pl.BlockSpec((blk, DM), lambda i: (i, 0)),
        out_shape=_f32(M, DM),
    )(tab_map, sc128, sc16, old)


def _tc_merge(tpos, ttime, mem_time, t2pos, t2time, mailbox_time):
    def body(tp_ref, tt_ref, mt_ref, t2p_ref, t2t_ref, mbt_ref,
             o1_ref, o2_ref):
        o1_ref[...] = jnp.where(tp_ref[...] >= 0, tt_ref[...], mt_ref[...])
        o2_ref[...] = jnp.where(t2p_ref[...] >= 0, t2t_ref[...],
                                mbt_ref[...])

    blk = 4096
    spec = pl.BlockSpec((blk,), lambda i: (i,))
    return pl.pallas_call(
        body,
        grid=(NW * RANGE // blk,),
        in_specs=[spec] * 6,
        out_specs=[spec, spec],
        out_shape=[_f32(M), _f32(M)],
    )(tpos, ttime, mem_time, t2pos, t2time, mailbox_time)


def kernel(nodes, times, mem_data, mem_time, mailbox_mail, mailbox_time,
           nfeat, efeat, W_ih, W_hh, b_ih, b_hh, te_w, te_b, tl_w, tl_b,
           ln_g, ln_b, ep_src_w, ep_src_b, ep_dst_w, ep_dst_b,
           ep_out_w, ep_out_b):
    P = _pack_times(mem_time, mailbox_time)
    nodes2d = nodes.reshape(NW, GCH, 128)
    gp, gmem, gmail, gnf = _sc_gather(nodes2d, P, mem_data, mailbox_mail,
                                      nfeat)
    embed, proj = _tc_dense(
        gp, gmail, gmem, gnf, times,
        W_ih, W_hh, b_ih.reshape(1, DE), b_hh.reshape(1, DE),
        te_w.reshape(1, DT), te_b.reshape(1, DT),
        tl_w.reshape(1, DE), tl_b.reshape(1, DE),
        ln_g.reshape(1, DE), ln_b.reshape(1, DE))
    pos, neg = _tc_scores(proj, ep_src_w, ep_src_b.reshape(1, DE),
                          ep_dst_w, ep_dst_b.reshape(1, DE),
                          ep_out_w.reshape(1, DE), ep_out_b.reshape(1, 1))

    mail_ts = gp[:, 1]
    upd2d = nodes[:L2].reshape(L2 // 1024, 1024)
    mts2d = mail_ts[:L2].reshape(L2 // 1024, 1024)
    dst2d = nodes[S:L2].reshape(S // 1024, 1024)
    tdst2d = times[S:L2].reshape(S // 1024, 1024)
    (scat_mem, mailnew, tpos, ttime, t2pos, t2time) = _sc_mem_update(
        upd2d, mts2d, dst2d, tdst2d, embed)
    scat_mail128, scat_mail16 = _sc_mail_update(mailnew, efeat, t2pos)
    tposf = tpos.reshape(NW * RANGE)
    t2posf = t2pos.reshape(NW * RANGE)
    new_mem_data = _tc_merge_rows(tposf.reshape(-1, 1), scat_mem,
                                  mem_data, DE)
    new_mailbox_mail = _tc_merge_mail(t2posf.reshape(-1, 1), scat_mail128,
                                      scat_mail16, mailbox_mail)
    new_mem_time, new_mailbox_time = _tc_merge(
        tposf, ttime.reshape(NW * RANGE), mem_time,
        t2posf, t2time.reshape(NW * RANGE), mailbox_time)

    return (pos[:, None], neg[:, None], new_mem_data, new_mem_time,
            new_mailbox_mail, new_mailbox_time)
